# Initial kernel scaffold; baseline (speedup 1.0000x reference)
import jax
import jax.numpy as jnp
from jax.experimental import pallas as pl


def kernel(x, edge_index, edge_attr, W1, att_src1, att_dst1, We1, att_e1, b1, W2, att_src2, att_dst2, We2, att_e2, b2, Wp1, bp1, Wp2, bp2):
    raise NotImplementedError("write your pallas kernel here")


# speedup vs baseline: 49.8907x; 49.8907x over previous
import functools

import jax
import jax.numpy as jnp
from jax import lax
from jax.experimental import pallas as pl
from jax.experimental.pallas import tpu as pltpu
from jax.experimental.pallas import tpu_sc as plsc

NC = 2
NS = 16
NW = NC * NS
K = 128

_GDN = lax.GatherDimensionNumbers(
    offset_dims=(), collapsed_slice_dims=(0,), start_index_map=(0,))


def _bcast_lane(v, i):
  idx = jnp.full((16, 1), i, jnp.int32)
  return lax.gather(v, idx, _GDN, (1,),
                    mode=lax.GatherScatterMode.PROMISE_IN_BOUNDS)


def _round_up(a, b):
  return (a + b - 1) // b * b


def _make_layer_pass(n_pad, et):
  chunks = et // K
  zrows = n_pad // NS
  mesh = plsc.VectorSubcoreMesh(core_axis_name="c", subcore_axis_name="s")

  @functools.partial(
      pl.kernel,
      out_type=[
          jax.ShapeDtypeStruct((NC, n_pad, 16), jnp.float32),
          jax.ShapeDtypeStruct((NC, n_pad), jnp.float32),
      ],
      mesh=mesh,
      scratch_types=[
          pltpu.VMEM((K,), jnp.int32),
          pltpu.VMEM((K,), jnp.int32),
          pltpu.VMEM((K,), jnp.float32),
          pltpu.VMEM((K, 32), jnp.float32),
          pltpu.VMEM((K, 32), jnp.float32),
          pltpu.VMEM((K, 16), jnp.float32),
          pltpu.VMEM((K,), jnp.float32),
          pltpu.VMEM((16,), jnp.float32),
          pltpu.VMEM_SHARED((n_pad, 16), jnp.float32),
          pltpu.VMEM_SHARED((n_pad,), jnp.float32),
          pltpu.SemaphoreType.DMA,
      ],
      compiler_params=pltpu.CompilerParams(needs_layout_passes=False, use_tc_tiling_on_sc=False),
  )
  def kern(tab, srce, dste, eae, cvec, num_out, den_out,
           srcv, dstv, eav, rs, rd, numb, exb, cbuf, num_acc, den_acc, sem):
    c = lax.axis_index("c")
    s = lax.axis_index("s")
    wid = c * NS + s

    zeros16 = jnp.zeros((16,), jnp.float32)

    @pl.loop(0, K)
    def _(i):
      numb[i, :] = zeros16

    @pl.loop(0, K // 16)
    def _(i):
      exb[pl.ds(i * 16, 16)] = zeros16

    zbase = s * zrows

    @pl.loop(0, zrows // K)
    def _(z):
      off = zbase + z * K
      pltpu.sync_copy(numb, num_acc.at[pl.ds(off, K)])
      pltpu.sync_copy(exb, den_acc.at[pl.ds(off, K)])

    plsc.subcore_barrier()

    pltpu.sync_copy(cvec, cbuf)
    cv = cbuf[...]
    ebase = wid * et

    @pl.loop(0, chunks)
    def _(q):
      base = ebase + q * K
      d1 = pltpu.async_copy(srce.at[pl.ds(base, K)], srcv, sem)
      d2 = pltpu.async_copy(dste.at[pl.ds(base, K)], dstv, sem)
      d3 = pltpu.async_copy(eae.at[pl.ds(base, K)], eav, sem)
      d1.wait()
      d2.wait()
      d3.wait()
      g1 = pltpu.async_copy(tab.at[srcv], rs, sem)
      g2 = pltpu.async_copy(tab.at[dstv], rd, sem)
      g1.wait()
      g2.wait()
      for g in range(K // 16):
        rows = lax.iota(jnp.int32, 16) + (g * 16)
        asrc = plsc.load_gather(rs, [rows, jnp.full((16,), 16, jnp.int32)])
        adst = plsc.load_gather(rd, [rows, jnp.full((16,), 17, jnp.int32)])
        ea = eav[pl.ds(g * 16, 16)]
        al = asrc + adst + cv * ea
        al = jnp.where(al >= 0, al, al * jnp.float32(0.2))
        ex = jnp.exp(al)
        exb[pl.ds(g * 16, 16)] = ex
        for i in range(16):
          e = g * 16 + i
          numb[e, :] = rs[e, pl.ds(0, 16)] * _bcast_lane(ex, i)
      pltpu.sync_copy(numb, num_acc.at[dstv], add=True)
      pltpu.sync_copy(exb, den_acc.at[dstv], add=True)

    plsc.subcore_barrier()

    pltpu.sync_copy(num_acc.at[pl.ds(zbase, zrows)],
                    num_out.at[c, pl.ds(zbase, zrows)])
    pltpu.sync_copy(den_acc.at[pl.ds(zbase, zrows)],
                    den_out.at[c, pl.ds(zbase, zrows)])

  return kern


def _make_pred_pass(e_pad, et):
  chunks = et // K
  mesh = plsc.VectorSubcoreMesh(core_axis_name="c", subcore_axis_name="s")

  @functools.partial(
      pl.kernel,
      out_type=jax.ShapeDtypeStruct((e_pad,), jnp.float32),
      mesh=mesh,
      scratch_types=[
          pltpu.VMEM((K,), jnp.int32),
          pltpu.VMEM((K,), jnp.int32),
          pltpu.VMEM((K,), jnp.float32),
          pltpu.VMEM((K, 32), jnp.float32),
          pltpu.VMEM((K, 32), jnp.float32),
          pltpu.VMEM((K,), jnp.float32),
          pltpu.VMEM((16, 16), jnp.float32),
          pltpu.VMEM((16, 16), jnp.float32),
          pltpu.VMEM((16,), jnp.float32),
          pltpu.SemaphoreType.DMA,
      ],
      compiler_params=pltpu.CompilerParams(needs_layout_passes=False, use_tc_tiling_on_sc=False),
  )
  def kern(tab, srce, dste, eae, wtab, w2tab, bvec, out,
           srcv, dstv, eav, rs, rd, outb, wt, w2t, bv, sem):
    c = lax.axis_index("c")
    s = lax.axis_index("s")
    wid = c * NS + s
    pltpu.sync_copy(wtab, wt)
    pltpu.sync_copy(w2tab, w2t)
    pltpu.sync_copy(bvec, bv)
    bias = bv[...]
    ebase = wid * et

    @pl.loop(0, chunks)
    def _(q):
      base = ebase + q * K
      d1 = pltpu.async_copy(srce.at[pl.ds(base, K)], srcv, sem)
      d2 = pltpu.async_copy(dste.at[pl.ds(base, K)], dstv, sem)
      d3 = pltpu.async_copy(eae.at[pl.ds(base, K)], eav, sem)
      d1.wait()
      d2.wait()
      d3.wait()
      g1 = pltpu.async_copy(tab.at[srcv], rs, sem)
      g2 = pltpu.async_copy(tab.at[dstv], rd, sem)
      g1.wait()
      g2.wait()
      for g in range(K // 16):
        rows = lax.iota(jnp.int32, 16) + (g * 16)
        ea = eav[pl.ds(g * 16, 16)]
        y = jnp.zeros((16,), jnp.float32)
        for j in range(16):
          rs_col = plsc.load_gather(rs, [rows, jnp.full((16,), j, jnp.int32)])
          rd_col = plsc.load_gather(
              rd, [rows, jnp.full((16,), 16 + j, jnp.int32)])
          v = rs_col + rd_col + ea * wt[j, :]
          v = jnp.maximum(v, jnp.float32(0.0))
          y = y + v * w2t[j, :]
        outv = jnp.maximum(y + bias, jnp.float32(0.0))
        outb[pl.ds(g * 16, 16)] = outv
      pltpu.sync_copy(outb, out.at[pl.ds(base, K)])

  return kern


def _tc1_body(x_ref, w_ref, asr_ref, adr_ref, we_ref, ae_ref, t_ref, c_ref):
  xb = x_ref[...]
  w = w_ref[...]
  h = (xb[:, 0:1] * w[0:1, :] + xb[:, 1:2] * w[1:2, :]
       + xb[:, 2:3] * w[2:3, :])
  asrc = jnp.sum(h * asr_ref[...], axis=1, keepdims=True)
  adst = jnp.sum(h * adr_ref[...], axis=1, keepdims=True)
  pad = jnp.zeros((h.shape[0], 14), jnp.float32)
  t_ref[...] = jnp.concatenate([h, asrc, adst, pad], axis=1)
  cval = jnp.sum(we_ref[...] * ae_ref[...])
  c_ref[...] = jnp.zeros((1, 16), jnp.float32) + cval


def _tc2_body(n_ref, d_ref, b_ref, w_ref, asr_ref, adr_ref, we_ref, ae_ref,
              t_ref, c_ref):
  num = n_ref[0] + n_ref[1]
  den = d_ref[0] + d_ref[1]
  h = num / (den[:, None] + jnp.float32(1e-16)) + b_ref[...]
  h = jnp.maximum(h, jnp.float32(0.0))
  g = jnp.dot(h, w_ref[...], preferred_element_type=jnp.float32)
  asrc = jnp.sum(g * asr_ref[...], axis=1, keepdims=True)
  adst = jnp.sum(g * adr_ref[...], axis=1, keepdims=True)
  pad = jnp.zeros((g.shape[0], 14), jnp.float32)
  t_ref[...] = jnp.concatenate([g, asrc, adst, pad], axis=1)
  cval = jnp.sum(we_ref[...] * ae_ref[...])
  c_ref[...] = jnp.zeros((1, 16), jnp.float32) + cval


def _tc3_body(n_ref, d_ref, b_ref, wpa_ref, wpb_ref, bp_ref, t_ref):
  num = n_ref[0] + n_ref[1]
  den = d_ref[0] + d_ref[1]
  h = num / (den[:, None] + jnp.float32(1e-16)) + b_ref[...]
  h = jnp.maximum(h, jnp.float32(0.0))
  psrc = jnp.dot(h, wpa_ref[...], preferred_element_type=jnp.float32)
  psrc = psrc + bp_ref[...]
  pdst = jnp.dot(h, wpb_ref[...], preferred_element_type=jnp.float32)
  t_ref[...] = jnp.concatenate([psrc, pdst], axis=1)


def kernel(x, edge_index, edge_attr, W1, att_src1, att_dst1, We1, att_e1, b1,
           W2, att_src2, att_dst2, We2, att_e2, b2, Wp1, bp1, Wp2, bp2):
  N = x.shape[0]
  E = edge_index.shape[1]
  n_pad = _round_up(N + 1, NS * K)
  e_pad = _round_up(E, NW * K)
  et = e_pad // NW
  nb = 16
  bn = n_pad // nb

  f32 = jnp.float32
  src = edge_index[0].astype(jnp.int32)
  dst = edge_index[1].astype(jnp.int32)
  src_p = jnp.concatenate([src, jnp.zeros((e_pad - E,), jnp.int32)])
  dst_p = jnp.concatenate([dst, jnp.full((e_pad - E,), N, jnp.int32)])
  ea_p = jnp.concatenate([edge_attr[:, 0].astype(f32),
                          jnp.zeros((e_pad - E,), f32)])
  x_p = jnp.concatenate([x.astype(f32), jnp.zeros((n_pad - N, 3), f32)])

  row116 = lambda a: a.astype(f32).reshape(1, 16)

  tc1 = pl.pallas_call(
      _tc1_body,
      out_shape=[jax.ShapeDtypeStruct((n_pad, 32), f32),
                 jax.ShapeDtypeStruct((1, 16), f32)],
      grid=(nb,),
      in_specs=[
          pl.BlockSpec((bn, 3), lambda i: (i, 0)),
          pl.BlockSpec((3, 16), lambda i: (0, 0)),
          pl.BlockSpec((1, 16), lambda i: (0, 0)),
          pl.BlockSpec((1, 16), lambda i: (0, 0)),
          pl.BlockSpec((1, 16), lambda i: (0, 0)),
          pl.BlockSpec((1, 16), lambda i: (0, 0)),
      ],
      out_specs=[
          pl.BlockSpec((bn, 32), lambda i: (i, 0)),
          pl.BlockSpec((1, 16), lambda i: (0, 0)),
      ],
  )
  t1, c1 = tc1(x_p, W1.astype(f32), row116(att_src1), row116(att_dst1),
               row116(We1), row116(att_e1))

  layer = _make_layer_pass(n_pad, et)
  num1, den1 = layer(t1, src_p, dst_p, ea_p, c1.reshape(16))

  tc2 = pl.pallas_call(
      _tc2_body,
      out_shape=[jax.ShapeDtypeStruct((n_pad, 32), f32),
                 jax.ShapeDtypeStruct((1, 16), f32)],
      grid=(nb,),
      in_specs=[
          pl.BlockSpec((2, bn, 16), lambda i: (0, i, 0)),
          pl.BlockSpec((2, bn), lambda i: (0, i)),
          pl.BlockSpec((1, 16), lambda i: (0, 0)),
          pl.BlockSpec((16, 16), lambda i: (0, 0)),
          pl.BlockSpec((1, 16), lambda i: (0, 0)),
          pl.BlockSpec((1, 16), lambda i: (0, 0)),
          pl.BlockSpec((1, 16), lambda i: (0, 0)),
          pl.BlockSpec((1, 16), lambda i: (0, 0)),
      ],
      out_specs=[
          pl.BlockSpec((bn, 32), lambda i: (i, 0)),
          pl.BlockSpec((1, 16), lambda i: (0, 0)),
      ],
  )
  t2, c2 = tc2(num1, den1, row116(b1), W2.astype(f32), row116(att_src2),
               row116(att_dst2), row116(We2), row116(att_e2))

  num2, den2 = layer(t2, src_p, dst_p, ea_p, c2.reshape(16))

  tc3 = pl.pallas_call(
      _tc3_body,
      out_shape=jax.ShapeDtypeStruct((n_pad, 32), f32),
      grid=(nb,),
      in_specs=[
          pl.BlockSpec((2, bn, 16), lambda i: (0, i, 0)),
          pl.BlockSpec((2, bn), lambda i: (0, i)),
          pl.BlockSpec((1, 16), lambda i: (0, 0)),
          pl.BlockSpec((16, 16), lambda i: (0, 0)),
          pl.BlockSpec((16, 16), lambda i: (0, 0)),
          pl.BlockSpec((1, 16), lambda i: (0, 0)),
      ],
      out_specs=pl.BlockSpec((bn, 32), lambda i: (i, 0)),
  )
  t3 = tc3(num2, den2, row116(b2), Wp1[0:16].astype(f32),
           Wp1[16:32].astype(f32), row116(bp1))

  wtab = jnp.broadcast_to(Wp1[32].astype(f32)[:, None], (16, 16))
  w2tab = jnp.broadcast_to(Wp2[:, 0].astype(f32)[:, None], (16, 16))
  bvec = jnp.broadcast_to(bp2.astype(f32), (16,))

  pred = _make_pred_pass(e_pad, et)
  outp = pred(t3, src_p, dst_p, ea_p, wtab, w2tab, bvec)

  return outp[:E].reshape(E, 1)


# speedup vs baseline: 65.2363x; 1.3076x over previous
import functools

import jax
import jax.numpy as jnp
from jax import lax
from jax.experimental import pallas as pl
from jax.experimental.pallas import tpu as pltpu
from jax.experimental.pallas import tpu_sc as plsc

NC = 2
NS = 16
NW = NC * NS
K = 128

_GDN = lax.GatherDimensionNumbers(
    offset_dims=(), collapsed_slice_dims=(0,), start_index_map=(0,))


def _bcast_lane(v, i):
  idx = jnp.full((16, 1), i, jnp.int32)
  return lax.gather(v, idx, _GDN, (1,),
                    mode=lax.GatherScatterMode.PROMISE_IN_BOUNDS)


def _round_up(a, b):
  return (a + b - 1) // b * b


def _make_layer_pass(n_pad, et):
  chunks = et // K
  zrows = n_pad // NS
  mesh = plsc.VectorSubcoreMesh(core_axis_name="c", subcore_axis_name="s")

  @functools.partial(
      pl.kernel,
      out_type=[
          jax.ShapeDtypeStruct((NC, n_pad, 16), jnp.float32),
          jax.ShapeDtypeStruct((NC, n_pad), jnp.float32),
      ],
      mesh=mesh,
      scratch_types=[
          pltpu.VMEM((2, K), jnp.int32),
          pltpu.VMEM((2, K), jnp.int32),
          pltpu.VMEM((2, K), jnp.float32),
          pltpu.VMEM((4, K, 32), jnp.float32),
          pltpu.VMEM((K, 16), jnp.float32),
          pltpu.VMEM((K,), jnp.float32),
          pltpu.VMEM((16,), jnp.float32),
          pltpu.VMEM_SHARED((n_pad, 16), jnp.float32),
          pltpu.VMEM_SHARED((n_pad,), jnp.float32),
          pltpu.SemaphoreType.DMA,
          pltpu.SemaphoreType.DMA,
          pltpu.SemaphoreType.DMA,
          pltpu.SemaphoreType.DMA,
          pltpu.SemaphoreType.DMA,
      ],
      compiler_params=pltpu.CompilerParams(needs_layout_passes=False, use_tc_tiling_on_sc=False),
  )
  def kern(tab, srce, dste, eae, cvec, num_out, den_out,
           srcv, dstv, eav, rsd, numb, exb, cbuf, num_acc, den_acc,
           sl0, sl1, sg0, sg1, ss):
    c = lax.axis_index("c")
    s = lax.axis_index("s")
    wid = c * NS + s
    sls = (sl0, sl1)
    sgs = (sg0, sg1)

    zeros16 = jnp.zeros((16,), jnp.float32)

    @pl.loop(0, K)
    def _(i):
      numb[i, :] = zeros16

    @pl.loop(0, K // 16)
    def _(i):
      exb[pl.ds(i * 16, 16)] = zeros16

    zbase = s * zrows

    @pl.loop(0, zrows // K)
    def _(z):
      off = zbase + z * K
      pltpu.sync_copy(numb, num_acc.at[pl.ds(off, K)])
      pltpu.sync_copy(exb, den_acc.at[pl.ds(off, K)])

    plsc.subcore_barrier()

    pltpu.sync_copy(cvec, cbuf)
    cv = cbuf[...]
    ebase = wid * et

    def lin_descs(q, p):
      base = ebase + jnp.minimum(q, chunks - 1) * K
      return [
          pltpu.make_async_copy(srce.at[pl.ds(base, K)], srcv.at[p], sls[p]),
          pltpu.make_async_copy(dste.at[pl.ds(base, K)], dstv.at[p], sls[p]),
          pltpu.make_async_copy(eae.at[pl.ds(base, K)], eav.at[p], sls[p]),
      ]

    def gat_descs(p):
      return [
          pltpu.make_async_copy(tab.at[srcv.at[p]], rsd.at[2 * p], sgs[p]),
          pltpu.make_async_copy(tab.at[dstv.at[p]], rsd.at[2 * p + 1], sgs[p]),
      ]

    def sct_descs(p):
      return [
          pltpu.make_async_copy(numb, num_acc.at[dstv.at[p]], ss),
          pltpu.make_async_copy(exb, den_acc.at[dstv.at[p]], ss),
      ]

    def compute(p):
      rs = rsd.at[2 * p]
      rd = rsd.at[2 * p + 1]
      for g in range(K // 16):
        rows = lax.iota(jnp.int32, 16) + (g * 16)
        asrc = plsc.load_gather(rs, [rows, jnp.full((16,), 16, jnp.int32)])
        adst = plsc.load_gather(rd, [rows, jnp.full((16,), 17, jnp.int32)])
        ea = eav[p, pl.ds(g * 16, 16)]
        al = asrc + adst + cv * ea
        al = jnp.where(al >= 0, al, al * jnp.float32(0.2))
        ex = jnp.exp(al)
        exb[pl.ds(g * 16, 16)] = ex
        for i in range(16):
          e = g * 16 + i
          numb[e, :] = rs[e, pl.ds(0, 16)] * _bcast_lane(ex, i)

    def phase(q, p):
      o = 1 - p
      for d in gat_descs(p):
        d.wait()
      compute(p)
      sd = sct_descs(p)
      for d in sd:
        d.start(add=True)
      for d in lin_descs(q + 1, o):
        d.wait()
      for d in gat_descs(o):
        d.start()
      for d in sd:
        d.wait()
      for d in lin_descs(q + 2, p):
        d.start()

    for d in lin_descs(0, 0):
      d.start()
    for d in lin_descs(0, 0):
      d.wait()
    for d in gat_descs(0):
      d.start()
    for d in lin_descs(1, 1):
      d.start()

    @pl.loop(0, (chunks - 2) // 2)
    def _(qq):
      phase(2 * qq, 0)
      phase(2 * qq + 1, 1)

    phase(chunks - 2, 0)
    for d in gat_descs(1):
      d.wait()
    compute(1)
    sd = sct_descs(1)
    for d in sd:
      d.start(add=True)
    for d in sd:
      d.wait()
    for d in lin_descs(chunks - 1, 0):
      d.wait()

    plsc.subcore_barrier()

    pltpu.sync_copy(num_acc.at[pl.ds(zbase, zrows)],
                    num_out.at[c, pl.ds(zbase, zrows)])
    pltpu.sync_copy(den_acc.at[pl.ds(zbase, zrows)],
                    den_out.at[c, pl.ds(zbase, zrows)])

  return kern


def _make_pred_pass(e_pad, et):
  chunks = et // K
  mesh = plsc.VectorSubcoreMesh(core_axis_name="c", subcore_axis_name="s")

  @functools.partial(
      pl.kernel,
      out_type=jax.ShapeDtypeStruct((e_pad,), jnp.float32),
      mesh=mesh,
      scratch_types=[
          pltpu.VMEM((2, K), jnp.int32),
          pltpu.VMEM((2, K), jnp.int32),
          pltpu.VMEM((2, K), jnp.float32),
          pltpu.VMEM((4, K, 32), jnp.float32),
          pltpu.VMEM((K,), jnp.float32),
          pltpu.VMEM((16, 16), jnp.float32),
          pltpu.VMEM((16, 16), jnp.float32),
          pltpu.VMEM((16,), jnp.float32),
          pltpu.SemaphoreType.DMA,
          pltpu.SemaphoreType.DMA,
          pltpu.SemaphoreType.DMA,
          pltpu.SemaphoreType.DMA,
          pltpu.SemaphoreType.DMA,
      ],
      compiler_params=pltpu.CompilerParams(needs_layout_passes=False, use_tc_tiling_on_sc=False),
  )
  def kern(tab, srce, dste, eae, wtab, w2tab, bvec, out,
           srcv, dstv, eav, rsd, outb, wt, w2t, bv,
           sl0, sl1, sg0, sg1, ss):
    c = lax.axis_index("c")
    s = lax.axis_index("s")
    wid = c * NS + s
    sls = (sl0, sl1)
    sgs = (sg0, sg1)
    pltpu.sync_copy(wtab, wt)
    pltpu.sync_copy(w2tab, w2t)
    pltpu.sync_copy(bvec, bv)
    bias = bv[...]
    wvs = [wt[j, :] for j in range(16)]
    w2vs = [w2t[j, :] for j in range(16)]
    ebase = wid * et

    def lin_descs(q, p):
      base = ebase + jnp.minimum(q, chunks - 1) * K
      return [
          pltpu.make_async_copy(srce.at[pl.ds(base, K)], srcv.at[p], sls[p]),
          pltpu.make_async_copy(dste.at[pl.ds(base, K)], dstv.at[p], sls[p]),
          pltpu.make_async_copy(eae.at[pl.ds(base, K)], eav.at[p], sls[p]),
      ]

    def gat_descs(p):
      return [
          pltpu.make_async_copy(tab.at[srcv.at[p]], rsd.at[2 * p], sgs[p]),
          pltpu.make_async_copy(tab.at[dstv.at[p]], rsd.at[2 * p + 1], sgs[p]),
      ]

    def out_desc(q):
      base = ebase + jnp.minimum(q, chunks - 1) * K
      return pltpu.make_async_copy(outb, out.at[pl.ds(base, K)], ss)

    def compute(p):
      rs = rsd.at[2 * p]
      rd = rsd.at[2 * p + 1]
      for g in range(K // 16):
        rows = lax.iota(jnp.int32, 16) + (g * 16)
        ea = eav[p, pl.ds(g * 16, 16)]
        y = jnp.zeros((16,), jnp.float32)
        for j in range(16):
          rs_col = plsc.load_gather(rs, [rows, jnp.full((16,), j, jnp.int32)])
          rd_col = plsc.load_gather(
              rd, [rows, jnp.full((16,), 16 + j, jnp.int32)])
          v = rs_col + rd_col + ea * wvs[j]
          v = jnp.maximum(v, jnp.float32(0.0))
          y = y + v * w2vs[j]
        outv = jnp.maximum(y + bias, jnp.float32(0.0))
        outb[pl.ds(g * 16, 16)] = outv

    def phase(q, p):
      o = 1 - p
      for d in gat_descs(p):
        d.wait()
      compute(p)
      sd = out_desc(q)
      sd.start()
      for d in lin_descs(q + 1, o):
        d.wait()
      for d in gat_descs(o):
        d.start()
      sd.wait()
      for d in lin_descs(q + 2, p):
        d.start()

    for d in lin_descs(0, 0):
      d.start()
    for d in lin_descs(0, 0):
      d.wait()
    for d in gat_descs(0):
      d.start()
    for d in lin_descs(1, 1):
      d.start()

    @pl.loop(0, (chunks - 2) // 2)
    def _(qq):
      phase(2 * qq, 0)
      phase(2 * qq + 1, 1)

    phase(chunks - 2, 0)
    for d in gat_descs(1):
      d.wait()
    compute(1)
    sd = out_desc(chunks - 1)
    sd.start()
    sd.wait()
    for d in lin_descs(chunks - 1, 0):
      d.wait()

  return kern


def _tc1_body(x_ref, w_ref, asr_ref, adr_ref, we_ref, ae_ref, t_ref, c_ref):
  xb = x_ref[...]
  w = w_ref[...]
  h = (xb[:, 0:1] * w[0:1, :] + xb[:, 1:2] * w[1:2, :]
       + xb[:, 2:3] * w[2:3, :])
  asrc = jnp.sum(h * asr_ref[...], axis=1, keepdims=True)
  adst = jnp.sum(h * adr_ref[...], axis=1, keepdims=True)
  pad = jnp.zeros((h.shape[0], 14), jnp.float32)
  t_ref[...] = jnp.concatenate([h, asrc, adst, pad], axis=1)
  cval = jnp.sum(we_ref[...] * ae_ref[...])
  c_ref[...] = jnp.zeros((1, 16), jnp.float32) + cval


def _tc2_body(n_ref, d_ref, b_ref, w_ref, asr_ref, adr_ref, we_ref, ae_ref,
              t_ref, c_ref):
  num = n_ref[0] + n_ref[1]
  den = d_ref[0] + d_ref[1]
  h = num / (den[:, None] + jnp.float32(1e-16)) + b_ref[...]
  h = jnp.maximum(h, jnp.float32(0.0))
  g = jnp.dot(h, w_ref[...], preferred_element_type=jnp.float32)
  asrc = jnp.sum(g * asr_ref[...], axis=1, keepdims=True)
  adst = jnp.sum(g * adr_ref[...], axis=1, keepdims=True)
  pad = jnp.zeros((g.shape[0], 14), jnp.float32)
  t_ref[...] = jnp.concatenate([g, asrc, adst, pad], axis=1)
  cval = jnp.sum(we_ref[...] * ae_ref[...])
  c_ref[...] = jnp.zeros((1, 16), jnp.float32) + cval


def _tc3_body(n_ref, d_ref, b_ref, wpa_ref, wpb_ref, bp_ref, t_ref):
  num = n_ref[0] + n_ref[1]
  den = d_ref[0] + d_ref[1]
  h = num / (den[:, None] + jnp.float32(1e-16)) + b_ref[...]
  h = jnp.maximum(h, jnp.float32(0.0))
  psrc = jnp.dot(h, wpa_ref[...], preferred_element_type=jnp.float32)
  psrc = psrc + bp_ref[...]
  pdst = jnp.dot(h, wpb_ref[...], preferred_element_type=jnp.float32)
  t_ref[...] = jnp.concatenate([psrc, pdst], axis=1)


def kernel(x, edge_index, edge_attr, W1, att_src1, att_dst1, We1, att_e1, b1,
           W2, att_src2, att_dst2, We2, att_e2, b2, Wp1, bp1, Wp2, bp2):
  N = x.shape[0]
  E = edge_index.shape[1]
  n_pad = _round_up(N + 1, NS * K)
  e_pad = _round_up(E, NW * K)
  et = e_pad // NW
  nb = 16
  bn = n_pad // nb

  f32 = jnp.float32
  src = edge_index[0].astype(jnp.int32)
  dst = edge_index[1].astype(jnp.int32)
  src_p = jnp.concatenate([src, jnp.zeros((e_pad - E,), jnp.int32)])
  dst_p = jnp.concatenate([dst, jnp.full((e_pad - E,), N, jnp.int32)])
  ea_p = jnp.concatenate([edge_attr[:, 0].astype(f32),
                          jnp.zeros((e_pad - E,), f32)])
  x_p = jnp.concatenate([x.astype(f32), jnp.zeros((n_pad - N, 3), f32)])

  row116 = lambda a: a.astype(f32).reshape(1, 16)

  tc1 = pl.pallas_call(
      _tc1_body,
      out_shape=[jax.ShapeDtypeStruct((n_pad, 32), f32),
                 jax.ShapeDtypeStruct((1, 16), f32)],
      grid=(nb,),
      in_specs=[
          pl.BlockSpec((bn, 3), lambda i: (i, 0)),
          pl.BlockSpec((3, 16), lambda i: (0, 0)),
          pl.BlockSpec((1, 16), lambda i: (0, 0)),
          pl.BlockSpec((1, 16), lambda i: (0, 0)),
          pl.BlockSpec((1, 16), lambda i: (0, 0)),
          pl.BlockSpec((1, 16), lambda i: (0, 0)),
      ],
      out_specs=[
          pl.BlockSpec((bn, 32), lambda i: (i, 0)),
          pl.BlockSpec((1, 16), lambda i: (0, 0)),
      ],
  )
  t1, c1 = tc1(x_p, W1.astype(f32), row116(att_src1), row116(att_dst1),
               row116(We1), row116(att_e1))

  layer = _make_layer_pass(n_pad, et)
  num1, den1 = layer(t1, src_p, dst_p, ea_p, c1.reshape(16))

  tc2 = pl.pallas_call(
      _tc2_body,
      out_shape=[jax.ShapeDtypeStruct((n_pad, 32), f32),
                 jax.ShapeDtypeStruct((1, 16), f32)],
      grid=(nb,),
      in_specs=[
          pl.BlockSpec((2, bn, 16), lambda i: (0, i, 0)),
          pl.BlockSpec((2, bn), lambda i: (0, i)),
          pl.BlockSpec((1, 16), lambda i: (0, 0)),
          pl.BlockSpec((16, 16), lambda i: (0, 0)),
          pl.BlockSpec((1, 16), lambda i: (0, 0)),
          pl.BlockSpec((1, 16), lambda i: (0, 0)),
          pl.BlockSpec((1, 16), lambda i: (0, 0)),
          pl.BlockSpec((1, 16), lambda i: (0, 0)),
      ],
      out_specs=[
          pl.BlockSpec((bn, 32), lambda i: (i, 0)),
          pl.BlockSpec((1, 16), lambda i: (0, 0)),
      ],
  )
  t2, c2 = tc2(num1, den1, row116(b1), W2.astype(f32), row116(att_src2),
               row116(att_dst2), row116(We2), row116(att_e2))

  num2, den2 = layer(t2, src_p, dst_p, ea_p, c2.reshape(16))

  tc3 = pl.pallas_call(
      _tc3_body,
      out_shape=jax.ShapeDtypeStruct((n_pad, 32), f32),
      grid=(nb,),
      in_specs=[
          pl.BlockSpec((2, bn, 16), lambda i: (0, i, 0)),
          pl.BlockSpec((2, bn), lambda i: (0, i)),
          pl.BlockSpec((1, 16), lambda i: (0, 0)),
          pl.BlockSpec((16, 16), lambda i: (0, 0)),
          pl.BlockSpec((16, 16), lambda i: (0, 0)),
          pl.BlockSpec((1, 16), lambda i: (0, 0)),
      ],
      out_specs=pl.BlockSpec((bn, 32), lambda i: (i, 0)),
  )
  t3 = tc3(num2, den2, row116(b2), Wp1[0:16].astype(f32),
           Wp1[16:32].astype(f32), row116(bp1))

  wtab = jnp.broadcast_to(Wp1[32].astype(f32)[:, None], (16, 16))
  w2tab = jnp.broadcast_to(Wp2[:, 0].astype(f32)[:, None], (16, 16))
  bvec = jnp.broadcast_to(bp2.astype(f32), (16,))

  pred = _make_pred_pass(e_pad, et)
  outp = pred(t3, src_p, dst_p, ea_p, wtab, w2tab, bvec)

  return outp[:E].reshape(E, 1)
